# baseline (speedup 1.0000x reference)
import jax
import jax.numpy as jnp
from jax.experimental import pallas as pl
from jax.experimental.pallas import tpu as pltpu

_CHUNK = 512
_NBUF = 4


def _router_block(x_hbm, w_ref, noise_ref, o_ref, xbuf, sems):
    i = pl.program_id(0)
    nchunks = pl.num_programs(0)

    @pl.when(i == 0)
    def _prime():
        for b in range(_NBUF):
            pltpu.make_async_copy(
                x_hbm.at[pl.ds(b * _CHUNK, _CHUNK), :],
                xbuf.at[b],
                sems.at[b],
            ).start()

    slot = jax.lax.rem(i, _NBUF)
    pltpu.make_async_copy(
        x_hbm.at[pl.ds(i * _CHUNK, _CHUNK), :],
        xbuf.at[slot],
        sems.at[slot],
    ).wait()

    acc = jax.lax.dot_general(
        xbuf[slot].astype(jnp.bfloat16),
        w_ref[...].astype(jnp.bfloat16),
        dimension_numbers=(((1,), (1,)), ((), ())),
        preferred_element_type=jnp.float32,
    )
    o_ref[...] = acc + noise_ref[...]

    nxt = i + _NBUF

    @pl.when(nxt < nchunks)
    def _refill():
        pltpu.make_async_copy(
            x_hbm.at[pl.ds(nxt * _CHUNK, _CHUNK), :],
            xbuf.at[slot],
            sems.at[slot],
        ).start()


@jax.jit
def kernel(x, W, noise):
    tokens, d_model = x.shape
    n_experts = W.shape[0]
    grid = (tokens // _CHUNK,)
    return pl.pallas_call(
        _router_block,
        grid=grid,
        in_specs=[
            pl.BlockSpec(memory_space=pl.ANY),
            pl.BlockSpec((n_experts, d_model), lambda i: (0, 0)),
            pl.BlockSpec((_CHUNK, n_experts), lambda i: (i, 0)),
        ],
        out_specs=pl.BlockSpec((_CHUNK, n_experts), lambda i: (i, 0)),
        out_shape=jax.ShapeDtypeStruct((tokens, n_experts), jnp.float32),
        scratch_shapes=[
            pltpu.VMEM((_NBUF, _CHUNK, d_model), jnp.float32),
            pltpu.SemaphoreType.DMA((_NBUF,)),
        ],
        compiler_params=pltpu.CompilerParams(
            dimension_semantics=("arbitrary",),
        ),
    )(x, W, noise)


# speedup vs baseline: 1.0725x; 1.0725x over previous
import jax
import jax.numpy as jnp
from jax.experimental import pallas as pl
from jax.experimental.pallas import tpu as pltpu

_BM = 512
_NSPLIT = 4
_SUB = _BM // _NSPLIT


def _router_block(*refs):
    x_refs = refs[:_NSPLIT]
    w_ref, noise_ref, o_ref = refs[_NSPLIT:]
    wb = w_ref[...].astype(jnp.bfloat16)
    dims = (((1,), (1,)), ((), ()))
    for s in range(_NSPLIT):
        acc = jax.lax.dot_general(
            x_refs[s][...].astype(jnp.bfloat16), wb, dimension_numbers=dims,
            preferred_element_type=jnp.float32,
        )
        o_ref[s * _SUB:(s + 1) * _SUB, :] = (
            acc + noise_ref[s * _SUB:(s + 1) * _SUB, :]
        )


def _x_spec(s):
    return pl.BlockSpec((_SUB, 4096), lambda i, s=s: (_NSPLIT * i + s, 0))


@jax.jit
def kernel(x, W, noise):
    tokens, d_model = x.shape
    n_experts = W.shape[0]
    grid = (tokens // _BM,)
    return pl.pallas_call(
        _router_block,
        grid=grid,
        in_specs=[_x_spec(s) for s in range(_NSPLIT)] + [
            pl.BlockSpec((n_experts, d_model), lambda i: (0, 0)),
            pl.BlockSpec((_BM, n_experts), lambda i: (i, 0)),
        ],
        out_specs=pl.BlockSpec((_BM, n_experts), lambda i: (i, 0)),
        out_shape=jax.ShapeDtypeStruct((tokens, n_experts), jnp.float32),
        compiler_params=pltpu.CompilerParams(
            dimension_semantics=("arbitrary",),
        ),
    )(*([x] * _NSPLIT), W, noise)
